# Initial kernel scaffold; baseline (speedup 1.0000x reference)
import jax
import jax.numpy as jnp
from jax.experimental import pallas as pl


def kernel(token_ids, token_emb, pos_emb):
    raise NotImplementedError("write your pallas kernel here")


# speedup vs baseline: 1.1765x; 1.1765x over previous
import functools

import jax
import jax.numpy as jnp
from jax import lax
from jax.experimental import pallas as pl
from jax.experimental.pallas import tpu as pltpu
from jax.experimental.pallas import tpu_sc as plsc

_NC = 2
_NS = 16
_NW = _NC * _NS
_PC = 32
_L = 16


@functools.lru_cache(maxsize=None)
def _build_sc_call(B, T, D, V, PMAX):
    PW = T // _NW
    NPC = PW // _PC
    NCH = NPC * B
    VPR = D // _L

    mesh = plsc.VectorSubcoreMesh(core_axis_name="c", subcore_axis_name="s")

    @functools.partial(
        pl.kernel,
        mesh=mesh,
        out_type=jax.ShapeDtypeStruct((B * T, D), jnp.float32),
        scratch_types=[
            pltpu.VMEM((NCH, _PC), jnp.int32),
            pltpu.VMEM((_PC, D), jnp.float32),
            pltpu.VMEM((2, _PC, D), jnp.float32),
            pltpu.SemaphoreType.DMA,
            pltpu.SemaphoreType.DMA,
            pltpu.SemaphoreType.DMA,
            pltpu.SemaphoreType.DMA,
        ],
    )
    def sc_embed(idx_hbm, emb_hbm, pos_hbm, out_hbm, idx_v, pos_v, tok_v,
                 gsem0, gsem1, wsem0, wsem1):
        gsems = [gsem0, gsem1]
        wsems = [wsem0, wsem1]
        wid = lax.axis_index("s") * _NC + lax.axis_index("c")
        pos0 = wid * PW

        pltpu.sync_copy(idx_hbm.at[wid], idx_v)

        def start_gather(k, buf):
            return pltpu.async_copy(emb_hbm.at[idx_v.at[k]], tok_v.at[buf],
                                    gsems[buf])

        def start_write(k, buf):
            pc, b = divmod(k, B)
            base = b * T + pos0 + pc * _PC
            return pltpu.async_copy(tok_v.at[buf], out_hbm.at[pl.ds(base, _PC)],
                                    wsems[buf])

        def add_pos(buf):
            def row_body(r, carry):
                for v in range(VPR):
                    sl = pl.ds(v * _L, _L)
                    tok_v[buf, r, sl] = tok_v[buf, r, sl] + pos_v[r, sl]
                return carry
            lax.fori_loop(0, _PC, row_body, 0)

        g_handles = [None, None]
        w_handles = [None, None]
        g_handles[0] = start_gather(0, 0)
        for k in range(NCH):
            pc, b = divmod(k, B)
            buf = k % 2
            if b == 0:
                pltpu.sync_copy(pos_hbm.at[pl.ds(pos0 + pc * _PC, _PC)], pos_v)
            if k + 1 < NCH:
                nbuf = (k + 1) % 2
                if w_handles[nbuf] is not None:
                    w_handles[nbuf].wait()
                g_handles[nbuf] = start_gather(k + 1, nbuf)
            g_handles[buf].wait()
            add_pos(buf)
            w_handles[buf] = start_write(k, buf)
        w_handles[0].wait()
        w_handles[1].wait()

    return sc_embed


def kernel(token_ids, token_emb, pos_emb):
    B, T = token_ids.shape
    V, D = token_emb.shape
    PMAX = pos_emb.shape[0]
    PW = T // _NW
    NPC = PW // _PC

    ids = token_ids.astype(jnp.int32)
    idx = (ids.reshape(B, _NW, NPC, _PC)
              .transpose(1, 2, 0, 3)
              .reshape(_NW, NPC * B, _PC))

    sc_embed = _build_sc_call(B, T, D, V, PMAX)
    out_flat = sc_embed(idx, token_emb, pos_emb)
    return out_flat.reshape(B, T, D)
